# Initial kernel scaffold; baseline (speedup 1.0000x reference)
import jax
import jax.numpy as jnp
from jax.experimental import pallas as pl


def kernel(inputs, token_table, pos_table):
    raise NotImplementedError("write your pallas kernel here")


# speedup vs baseline: 3.6989x; 3.6989x over previous
import functools

import jax
import jax.numpy as jnp
from jax import lax
from jax.experimental import pallas as pl
from jax.experimental.pallas import tpu as pltpu
from jax.experimental.pallas import tpu_sc as plsc

_LANES = 16


def _sc_geometry():
    try:
        info = plsc.get_sparse_core_info()
        return info.num_cores, info.num_subcores
    except Exception:
        return 2, 16


def kernel(inputs, token_table, pos_table):
    batch, seq = inputs.shape
    vocab, emb = token_table.shape
    n = batch * seq

    nc, ns = _sc_geometry()
    nw = nc * ns
    per_w = n // nw
    seqs_per_chunk = 4
    ch = seqs_per_chunk * seq
    n_ch = per_w // ch

    idx_flat = inputs.reshape(n).astype(jnp.int32)

    mesh = plsc.VectorSubcoreMesh(core_axis_name="c", subcore_axis_name="s")

    @functools.partial(
        pl.kernel,
        out_type=jax.ShapeDtypeStruct((n, emb), jnp.float32),
        mesh=mesh,
        scratch_types=[
            pltpu.VMEM((ch,), jnp.int32),
            pltpu.VMEM((ch, emb), jnp.float32),
            pltpu.VMEM((seq, emb), jnp.float32),
            pltpu.SemaphoreType.DMA,
        ],
        compiler_params=pltpu.CompilerParams(use_tc_tiling_on_sc=False),
    )
    def sc_kernel(idx_hbm, tok_hbm, pos_hbm, out_hbm, idx_v, rows_v, pos_v, sem):
        wid = lax.axis_index("s") * nc + lax.axis_index("c")
        base = wid * per_w
        pltpu.sync_copy(pos_hbm, pos_v)

        def chunk_body(c, carry):
            off = base + c * ch
            pltpu.sync_copy(idx_hbm.at[pl.ds(off, ch)], idx_v)
            pltpu.async_copy(tok_hbm.at[idx_v], rows_v, sem).wait()

            def s_body(s, carry2):
                for g in range(emb // _LANES):
                    sl = pl.ds(g * _LANES, _LANES)
                    p = pos_v[s, sl]
                    for rep in range(seqs_per_chunk):
                        plsc.addupdate(rows_v.at[rep * seq + s, sl], p)
                return carry2

            lax.fori_loop(0, seq, s_body, 0)
            pltpu.sync_copy(rows_v, out_hbm.at[pl.ds(off, ch)])
            return carry

        lax.fori_loop(0, n_ch, chunk_body, 0)

    out = sc_kernel(idx_flat, token_table, pos_table)
    return out.reshape(batch, seq, emb)
